# Initial kernel scaffold; baseline (speedup 1.0000x reference)
import jax
import jax.numpy as jnp
from jax.experimental import pallas as pl


def kernel(image_emb, tex, params, lap_idx, lap_val, up_idx, up_val, down_idx, down_val):
    raise NotImplementedError("write your pallas kernel here")


# speedup vs baseline: 10.0947x; 10.0947x over previous
import functools

import jax
import jax.numpy as jnp
from jax import lax
from jax.experimental import pallas as pl
from jax.experimental.pallas import tpu as pltpu
from jax.experimental.pallas import tpu_sc as plsc

_K = 6
_B = 4
_NW = 32
_LANES = 16


def _fp(f):
    return {3: 4, 6: 8}.get(f, f)


@functools.cache
def _make_spmm(ncols, nrows, C, deg, has_prev):
    epw = nrows * deg // _NW
    EC = min(96 if deg == 3 else 128, epw)
    assert EC % deg == 0 and epw % EC == 0 and EC % 8 == 0
    RC = EC // deg
    nch = epw // EC
    rows_pw = nrows // _NW
    G = C // _LANES
    assert C % _LANES == 0

    mesh = plsc.VectorSubcoreMesh(core_axis_name="c", subcore_axis_name="s",
                                  num_cores=2, num_subcores=16)
    scratch = [
        pltpu.VMEM((EC,), jnp.int32),
        pltpu.VMEM((EC,), jnp.float32),
        pltpu.VMEM((EC, C), jnp.float32),
        pltpu.VMEM((RC, C), jnp.float32),
    ]
    if has_prev:
        scratch.append(pltpu.VMEM((RC, C), jnp.float32))
    scratch.append(pltpu.SemaphoreType.DMA)

    def body(*refs):
        if has_prev:
            (x_hbm, col_hbm, val_hbm, prev_hbm, out_hbm,
             idx_v, val_v, gat_v, out_v, prev_v, sem) = refs
        else:
            (x_hbm, col_hbm, val_hbm, out_hbm,
             idx_v, val_v, gat_v, out_v, sem) = refs
        cid = lax.axis_index("c")
        sid = lax.axis_index("s")
        wid = sid * 2 + cid
        ebase = wid * epw
        rbase = wid * rows_pw

        def chunk(j, carry):
            e0 = ebase + j * EC
            r0 = rbase + j * RC
            pltpu.sync_copy(col_hbm.at[pl.ds(e0, EC)], idx_v)
            gcp = pltpu.async_copy(x_hbm.at[idx_v], gat_v, sem)
            pltpu.sync_copy(val_hbm.at[pl.ds(e0, EC)], val_v)
            if has_prev:
                pltpu.sync_copy(prev_hbm.at[pl.ds(r0, RC)], prev_v)
            gcp.wait()

            def row(r, c2):
                eb = r * deg
                w = plsc.load_gather(
                    val_v, [lax.broadcast_in_dim(eb, (16,), ())])
                acc = [w * gat_v[eb, pl.ds(g * 16, 16)] for g in range(G)]
                for d in range(1, deg):
                    wd = plsc.load_gather(
                        val_v, [lax.broadcast_in_dim(eb + d, (16,), ())])
                    for g in range(G):
                        acc[g] = acc[g] + wd * gat_v[eb + d, pl.ds(g * 16, 16)]
                for g in range(G):
                    res = acc[g]
                    if has_prev:
                        res = res + res - prev_v[r, pl.ds(g * 16, 16)]
                    out_v[r, pl.ds(g * 16, 16)] = res
                return c2

            lax.fori_loop(0, RC, row, 0)
            pltpu.sync_copy(out_v, out_hbm.at[pl.ds(r0, RC)])
            return carry

        lax.fori_loop(0, nch, chunk, 0)

    return pl.kernel(
        body,
        out_type=jax.ShapeDtypeStruct((nrows, C), jnp.float32),
        mesh=mesh,
        scratch_types=scratch,
        compiler_params=pltpu.CompilerParams(needs_layout_passes=False,
                                             use_tc_tiling_on_sc=False),
    )


def _spmm(x, col, val, nrows, deg, prev=None):
    ncols, C = x.shape
    f = _make_spmm(ncols, nrows, C, deg, prev is not None)
    if prev is None:
        return f(x, col, val)
    return f(x, col, val, prev)


def _mm(ts, wbd, bias, add_in, act):
    n, ci = ts[0].shape
    kk, _, co = wbd.shape
    bn = min(n, 2048)
    grid = n // bn
    in_specs = [pl.BlockSpec((bn, ci), lambda i: (i, 0)) for _ in ts]
    in_specs.append(pl.BlockSpec((kk, ci, co), lambda i: (0, 0, 0)))
    in_specs.append(pl.BlockSpec((1, co), lambda i: (0, 0)))
    args = list(ts) + [wbd, bias]
    if add_in is not None:
        in_specs.append(pl.BlockSpec((bn, co), lambda i: (i, 0)))
        args.append(add_in)

    def body(*refs):
        if add_in is not None:
            *t_refs, w_ref, b_ref, a_ref, o_ref = refs
        else:
            *t_refs, w_ref, b_ref, o_ref = refs
            a_ref = None
        acc = jnp.dot(t_refs[0][...], w_ref[0],
                      preferred_element_type=jnp.float32)
        for k in range(1, len(t_refs)):
            acc = acc + jnp.dot(t_refs[k][...], w_ref[k],
                                preferred_element_type=jnp.float32)
        acc = acc + b_ref[...]
        if a_ref is not None:
            acc = acc + a_ref[...]
        if act == "relu":
            acc = jnp.maximum(acc, 0.0)
        elif act == "tanh":
            acc = jnp.tanh(acc)
        o_ref[...] = acc

    return pl.pallas_call(
        body,
        grid=(grid,),
        in_specs=in_specs,
        out_specs=pl.BlockSpec((bn, co), lambda i: (i, 0)),
        out_shape=jax.ShapeDtypeStruct((n, co), jnp.float32),
    )(*args)


def _fc_tc(emb, w_t, b):
    dout = w_t.shape[1]

    def body(e_ref, w_ref, b_ref, o_ref):
        acc = jnp.dot(e_ref[...], w_ref[...],
                      preferred_element_type=jnp.float32)
        o_ref[...] = jnp.maximum(acc + b_ref[...], 0.0)

    return pl.pallas_call(
        body,
        out_shape=jax.ShapeDtypeStruct((8, dout), jnp.float32),
    )(emb, w_t, b)


def _prep_w(p, fin, cout):
    fp, cp = _fp(fin), _fp(cout)
    w = p["W"]
    core = w.reshape(cout, _K, fin).transpose(1, 2, 0)
    if fin == 6:
        h1 = jnp.pad(core[:, :3, :], ((0, 0), (0, 1), (0, cp - cout)))
        h2 = jnp.pad(core[:, 3:, :], ((0, 0), (0, 1), (0, cp - cout)))
        wp = jnp.concatenate([h1, h2], axis=1)
    else:
        wp = jnp.pad(core, ((0, 0), (0, fp - fin), (0, cp - cout)))
    eye = jnp.eye(_B, dtype=jnp.float32)
    wbd = jnp.einsum("ij,kfc->kifjc", eye, wp).reshape(_K, _B * fp, _B * cp)
    bias = jnp.pad(p["b"], (0, cp - cout))
    bias = jnp.tile(bias, _B).reshape(1, _B * cp)
    return wbd, bias


def _cheb(x, wb, col, val, n, act=None, add_in=None):
    wbd, bias = wb
    ts = [x]
    t1 = _spmm(x, col, val, n, 8)
    ts.append(t1)
    t0, t1_ = x, t1
    for _ in range(2, _K):
        t2 = _spmm(t1_, col, val, n, 8, prev=t0)
        ts.append(t2)
        t0, t1_ = t1_, t2
    return _mm(ts, wbd, bias, add_in, act)


def _res(x, p, col, val, n, cin, cout):
    if "shortcut" in p:
        sc = _cheb(x, _prep_w(p["shortcut"], cin, cout), col, val, n)
    else:
        sc = x
    h = _cheb(x, _prep_w(p["cheby1"], cin, cout), col, val, n, act="relu")
    return _cheb(h, _prep_w(p["cheby2"], cout, cout), col, val, n,
                 act="relu", add_in=sc)


def kernel(image_emb, tex, params, lap_idx, lap_val, up_idx, up_val,
           down_idx, down_val):
    ns = [32768, 8192, 2048, 512, 128]

    emb_p = jnp.pad(image_emb, ((0, 8 - _B), (0, 0)))
    w_t = params["fc"]["W"].T
    b = params["fc"]["b"].reshape(1, -1)
    h = _fc_tc(emb_p, w_t, b)[:_B]
    x = h.reshape(_B, ns[4], 32).transpose(1, 0, 2).reshape(ns[4], _B * 32)

    x = _spmm(x, up_idx[3][1], up_val[3], ns[3], 3)
    x = _res(x, params["dec0"], lap_idx[3][1], lap_val[3], ns[3], 32, 16)
    x = _spmm(x, up_idx[2][1], up_val[2], ns[2], 3)
    x = _res(x, params["dec1"], lap_idx[2][1], lap_val[2], ns[2], 16, 16)
    x = _spmm(x, up_idx[1][1], up_val[1], ns[1], 3)
    x = _res(x, params["dec2"], lap_idx[1][1], lap_val[1], ns[1], 16, 16)
    x = _spmm(x, up_idx[0][1], up_val[0], ns[0], 3)
    dec = _res(x, params["dec3"], lap_idx[0][1], lap_val[0], ns[0], 16, 3)

    t = tex.transpose(1, 0, 2)
    t = jnp.pad(t, ((0, 0), (0, 0), (0, 1))).reshape(ns[0], _B * 4)
    r = _res(t, params["ref0"], lap_idx[0][1], lap_val[0], ns[0], 3, 16)
    r = _spmm(r, down_idx[1], down_val, ns[1], 4)
    r = _res(r, params["ref1"], lap_idx[1][1], lap_val[1], ns[1], 16, 32)
    r = _res(r, params["ref2"], lap_idx[1][1], lap_val[1], ns[1], 32, 32)
    r = _spmm(r, up_idx[0][1], up_val[0], ns[0], 3)
    r = _res(r, params["ref3"], lap_idx[0][1], lap_val[0], ns[0], 32, 16)
    ref = _res(r, params["ref4"], lap_idx[0][1], lap_val[0], ns[0], 16, 3)

    cat = jnp.concatenate(
        [dec.reshape(ns[0], _B, 4), ref.reshape(ns[0], _B, 4)],
        axis=2).reshape(ns[0], _B * 8)
    out = _cheb(cat, _prep_w(params["comb"], 6, 3),
                lap_idx[0][1], lap_val[0], ns[0], act="tanh")
    out = out.reshape(ns[0], _B, 4).transpose(1, 0, 2)[:, :, :3]
    return out


# speedup vs baseline: 20.3439x; 2.0153x over previous
import functools

import jax
import jax.numpy as jnp
from jax import lax
from jax.experimental import pallas as pl
from jax.experimental.pallas import tpu as pltpu
from jax.experimental.pallas import tpu_sc as plsc

_K = 6
_B = 4
_NW = 32
_LANES = 16


def _fp(f):
    return {3: 4, 6: 8}.get(f, f)


@functools.cache
def _make_spmm(ncols, nrows, C, deg, has_prev):
    epw = nrows * deg // _NW
    EC = min(96 if deg == 3 else 128, epw)
    assert EC % deg == 0 and epw % EC == 0 and EC % 8 == 0
    RC = EC // deg
    nch = epw // EC
    assert nch == 1 or nch % 2 == 0
    rows_pw = nrows // _NW
    G = C // _LANES
    assert C % _LANES == 0
    NB = 2 if nch > 1 else 1

    mesh = plsc.VectorSubcoreMesh(core_axis_name="c", subcore_axis_name="s",
                                  num_cores=2, num_subcores=16)
    scratch = [
        pltpu.VMEM((epw,), jnp.int32),
        pltpu.VMEM((epw,), jnp.float32),
    ]
    scratch += [pltpu.VMEM((EC, C), jnp.float32) for _ in range(NB)]
    scratch += [pltpu.VMEM((RC, C), jnp.float32) for _ in range(NB)]
    if has_prev:
        scratch += [pltpu.VMEM((RC, C), jnp.float32) for _ in range(NB)]
    scratch += [pltpu.SemaphoreType.DMA for _ in range(2 * NB)]

    def body(*refs):
        if has_prev:
            x_hbm, col_hbm, val_hbm, prev_hbm, out_hbm = refs[:5]
            s = refs[5:]
        else:
            x_hbm, col_hbm, val_hbm, out_hbm = refs[:4]
            s = refs[4:]
        col_v, val_v = s[0], s[1]
        p = 2
        gat = s[p:p + NB]; p += NB
        outv = s[p:p + NB]; p += NB
        if has_prev:
            prevv = s[p:p + NB]; p += NB
        semg = s[p:p + NB]
        semo = s[p + NB:p + 2 * NB]

        wid = lax.axis_index("s") * 2 + lax.axis_index("c")
        ebase = wid * epw
        rbase = wid * rows_pw
        pltpu.sync_copy(col_hbm.at[pl.ds(ebase, epw)], col_v)
        pltpu.sync_copy(val_hbm.at[pl.ds(ebase, epw)], val_v)

        def fire(j, b):
            pltpu.async_copy(
                x_hbm.at[col_v.at[pl.ds(j * EC, EC)]], gat[b], semg[b])
            if has_prev:
                pltpu.async_copy(
                    prev_hbm.at[pl.ds(rbase + j * RC, RC)], prevv[b], semg[b])

        def wait_in(b):
            pltpu.make_async_copy(
                x_hbm.at[pl.ds(0, EC)], gat[b], semg[b]).wait()
            if has_prev:
                pltpu.make_async_copy(
                    prev_hbm.at[pl.ds(0, RC)], prevv[b], semg[b]).wait()

        def wait_out(b):
            pltpu.make_async_copy(
                outv[b], out_hbm.at[pl.ds(0, RC)], semo[b]).wait()

        def compute(j, b):
            nacc = 2 if (G <= 2 and deg >= 4) else 1

            def row(r, c2):
                ev = j * EC + r * deg
                eg = r * deg
                acc = [[None] * nacc for _ in range(G)]
                for d in range(deg):
                    wd = plsc.load_gather(
                        val_v, [lax.broadcast_in_dim(ev + d, (16,), ())])
                    a = d % nacc
                    for g in range(G):
                        term = wd * gat[b][eg + d, pl.ds(g * 16, 16)]
                        acc[g][a] = (term if acc[g][a] is None
                                     else acc[g][a] + term)
                for g in range(G):
                    res = acc[g][0]
                    for a in range(1, nacc):
                        res = res + acc[g][a]
                    if has_prev:
                        res = res + res - prevv[b][r, pl.ds(g * 16, 16)]
                    outv[b][r, pl.ds(g * 16, 16)] = res
                return c2

            lax.fori_loop(0, RC, row, 0)
            pltpu.async_copy(
                outv[b], out_hbm.at[pl.ds(rbase + j * RC, RC)], semo[b])

        if nch == 1:
            fire(0, 0)
            wait_in(0)
            compute(0, 0)
            wait_out(0)
        else:
            fire(0, 0)
            half = nch // 2

            def pair(pi, c):
                j = 2 * pi
                fire(j + 1, 1)
                wait_in(0)

                @pl.when(pi > 0)
                def _():
                    wait_out(0)
                compute(j, 0)

                @pl.when(pi < half - 1)
                def _():
                    fire(j + 2, 0)
                wait_in(1)

                @pl.when(pi > 0)
                def _():
                    wait_out(1)
                compute(j + 1, 1)
                return c

            lax.fori_loop(0, half, pair, 0)
            wait_out(0)
            wait_out(1)

    return pl.kernel(
        body,
        out_type=jax.ShapeDtypeStruct((nrows, C), jnp.float32),
        mesh=mesh,
        scratch_types=scratch,
        compiler_params=pltpu.CompilerParams(needs_layout_passes=False,
                                             use_tc_tiling_on_sc=False),
    )


def _spmm(x, col, val, nrows, deg, prev=None):
    ncols, C = x.shape
    f = _make_spmm(ncols, nrows, C, deg, prev is not None)
    if prev is None:
        return f(x, col, val)
    return f(x, col, val, prev)


def _mm(ts, wbd, bias, add_in, act):
    n, ci = ts[0].shape
    kk, _, co = wbd.shape
    bn = min(n, 2048)
    grid = n // bn
    in_specs = [pl.BlockSpec((bn, ci), lambda i: (i, 0)) for _ in ts]
    in_specs.append(pl.BlockSpec((kk, ci, co), lambda i: (0, 0, 0)))
    in_specs.append(pl.BlockSpec((1, co), lambda i: (0, 0)))
    args = list(ts) + [wbd, bias]
    if add_in is not None:
        in_specs.append(pl.BlockSpec((bn, co), lambda i: (i, 0)))
        args.append(add_in)

    def body(*refs):
        if add_in is not None:
            *t_refs, w_ref, b_ref, a_ref, o_ref = refs
        else:
            *t_refs, w_ref, b_ref, o_ref = refs
            a_ref = None
        acc = jnp.dot(t_refs[0][...], w_ref[0],
                      preferred_element_type=jnp.float32)
        for k in range(1, len(t_refs)):
            acc = acc + jnp.dot(t_refs[k][...], w_ref[k],
                                preferred_element_type=jnp.float32)
        acc = acc + b_ref[...]
        if a_ref is not None:
            acc = acc + a_ref[...]
        if act == "relu":
            acc = jnp.maximum(acc, 0.0)
        elif act == "tanh":
            acc = jnp.tanh(acc)
        o_ref[...] = acc

    return pl.pallas_call(
        body,
        grid=(grid,),
        in_specs=in_specs,
        out_specs=pl.BlockSpec((bn, co), lambda i: (i, 0)),
        out_shape=jax.ShapeDtypeStruct((n, co), jnp.float32),
    )(*args)


def _fc_tc(emb, w_t, b):
    dout = w_t.shape[1]

    def body(e_ref, w_ref, b_ref, o_ref):
        acc = jnp.dot(e_ref[...], w_ref[...],
                      preferred_element_type=jnp.float32)
        o_ref[...] = jnp.maximum(acc + b_ref[...], 0.0)

    return pl.pallas_call(
        body,
        out_shape=jax.ShapeDtypeStruct((8, dout), jnp.float32),
    )(emb, w_t, b)


def _prep_w(p, fin, cout):
    fp, cp = _fp(fin), _fp(cout)
    w = p["W"]
    core = w.reshape(cout, _K, fin).transpose(1, 2, 0)
    if fin == 6:
        h1 = jnp.pad(core[:, :3, :], ((0, 0), (0, 1), (0, cp - cout)))
        h2 = jnp.pad(core[:, 3:, :], ((0, 0), (0, 1), (0, cp - cout)))
        wp = jnp.concatenate([h1, h2], axis=1)
    else:
        wp = jnp.pad(core, ((0, 0), (0, fp - fin), (0, cp - cout)))
    eye = jnp.eye(_B, dtype=jnp.float32)
    wbd = jnp.einsum("ij,kfc->kifjc", eye, wp).reshape(_K, _B * fp, _B * cp)
    bias = jnp.pad(p["b"], (0, cp - cout))
    bias = jnp.tile(bias, _B).reshape(1, _B * cp)
    return wbd, bias


def _cheb_ts(x, col, val, n):
    ts = [x]
    t1 = _spmm(x, col, val, n, 8)
    ts.append(t1)
    t0, t1_ = x, t1
    for _ in range(2, _K):
        t2 = _spmm(t1_, col, val, n, 8, prev=t0)
        ts.append(t2)
        t0, t1_ = t1_, t2
    return ts


def _cheb(x, wb, col, val, n, act=None, add_in=None):
    ts = _cheb_ts(x, col, val, n)
    return _mm(ts, wb[0], wb[1], add_in, act)


def _res(x, p, col, val, n, cin, cout):
    ts = _cheb_ts(x, col, val, n)
    if "shortcut" in p:
        wbd, bias = _prep_w(p["shortcut"], cin, cout)
        sc = _mm(ts, wbd, bias, None, None)
    else:
        sc = x
    w1, b1 = _prep_w(p["cheby1"], cin, cout)
    h = _mm(ts, w1, b1, None, "relu")
    ts2 = _cheb_ts(h, col, val, n)
    w2, b2 = _prep_w(p["cheby2"], cout, cout)
    return _mm(ts2, w2, b2, sc, "relu")


def kernel(image_emb, tex, params, lap_idx, lap_val, up_idx, up_val,
           down_idx, down_val):
    ns = [32768, 8192, 2048, 512, 128]

    emb_p = jnp.pad(image_emb, ((0, 8 - _B), (0, 0)))
    w_t = params["fc"]["W"].T
    b = params["fc"]["b"].reshape(1, -1)
    h = _fc_tc(emb_p, w_t, b)[:_B]
    x = h.reshape(_B, ns[4], 32).transpose(1, 0, 2).reshape(ns[4], _B * 32)

    x = _spmm(x, up_idx[3][1], up_val[3], ns[3], 3)
    x = _res(x, params["dec0"], lap_idx[3][1], lap_val[3], ns[3], 32, 16)
    x = _spmm(x, up_idx[2][1], up_val[2], ns[2], 3)
    x = _res(x, params["dec1"], lap_idx[2][1], lap_val[2], ns[2], 16, 16)
    x = _spmm(x, up_idx[1][1], up_val[1], ns[1], 3)
    x = _res(x, params["dec2"], lap_idx[1][1], lap_val[1], ns[1], 16, 16)
    x = _spmm(x, up_idx[0][1], up_val[0], ns[0], 3)
    dec = _res(x, params["dec3"], lap_idx[0][1], lap_val[0], ns[0], 16, 3)

    t = tex.transpose(1, 0, 2)
    t = jnp.pad(t, ((0, 0), (0, 0), (0, 1))).reshape(ns[0], _B * 4)
    r = _res(t, params["ref0"], lap_idx[0][1], lap_val[0], ns[0], 3, 16)
    r = _spmm(r, down_idx[1], down_val, ns[1], 4)
    r = _res(r, params["ref1"], lap_idx[1][1], lap_val[1], ns[1], 16, 32)
    r = _res(r, params["ref2"], lap_idx[1][1], lap_val[1], ns[1], 32, 32)
    r = _spmm(r, up_idx[0][1], up_val[0], ns[0], 3)
    r = _res(r, params["ref3"], lap_idx[0][1], lap_val[0], ns[0], 32, 16)
    ref = _res(r, params["ref4"], lap_idx[0][1], lap_val[0], ns[0], 16, 3)

    cat = jnp.concatenate(
        [dec.reshape(ns[0], _B, 4), ref.reshape(ns[0], _B, 4)],
        axis=2).reshape(ns[0], _B * 8)
    out = _cheb(cat, _prep_w(params["comb"], 6, 3),
                lap_idx[0][1], lap_val[0], ns[0], act="tanh")
    out = out.reshape(ns[0], _B, 4).transpose(1, 0, 2)[:, :, :3]
    return out


# speedup vs baseline: 23.0143x; 1.1313x over previous
import functools

import jax
import jax.numpy as jnp
from jax import lax
from jax.experimental import pallas as pl
from jax.experimental.pallas import tpu as pltpu
from jax.experimental.pallas import tpu_sc as plsc

_K = 6
_B = 4
_NW = 32
_LANES = 16


def _fp(f):
    return {3: 4, 6: 8}.get(f, f)


@functools.cache
def _make_spmm(ncols, nrows, C, deg, has_prev):
    epw = nrows * deg // _NW
    EC = min(96 if deg == 3 else 128, epw)
    assert EC % deg == 0 and epw % EC == 0 and EC % 8 == 0
    RC = EC // deg
    nch = epw // EC
    rows_pw = nrows // _NW
    G = C // _LANES
    assert C % _LANES == 0
    NB = min(2 if C > 64 else 4, nch)
    assert nch % NB == 0

    mesh = plsc.VectorSubcoreMesh(core_axis_name="c", subcore_axis_name="s",
                                  num_cores=2, num_subcores=16)
    scratch = [
        pltpu.VMEM((epw,), jnp.int32),
        pltpu.VMEM((epw,), jnp.float32),
    ]
    scratch += [pltpu.VMEM((EC, C), jnp.float32) for _ in range(NB)]
    scratch += [pltpu.VMEM((RC, C), jnp.float32) for _ in range(NB)]
    if has_prev:
        scratch += [pltpu.VMEM((RC, C), jnp.float32) for _ in range(NB)]
    scratch += [pltpu.SemaphoreType.DMA for _ in range(2 * NB)]

    def body(*refs):
        if has_prev:
            x_hbm, col_hbm, val_hbm, prev_hbm, out_hbm = refs[:5]
            s = refs[5:]
        else:
            x_hbm, col_hbm, val_hbm, out_hbm = refs[:4]
            s = refs[4:]
        col_v, val_v = s[0], s[1]
        p = 2
        gat = s[p:p + NB]; p += NB
        outv = s[p:p + NB]; p += NB
        if has_prev:
            prevv = s[p:p + NB]; p += NB
        semg = s[p:p + NB]
        semo = s[p + NB:p + 2 * NB]

        wid = lax.axis_index("s") * 2 + lax.axis_index("c")
        ebase = wid * epw
        rbase = wid * rows_pw
        pltpu.sync_copy(col_hbm.at[pl.ds(ebase, epw)], col_v)
        pltpu.sync_copy(val_hbm.at[pl.ds(ebase, epw)], val_v)

        def fire(j, b):
            pltpu.async_copy(
                x_hbm.at[col_v.at[pl.ds(j * EC, EC)]], gat[b], semg[b])
            if has_prev:
                pltpu.async_copy(
                    prev_hbm.at[pl.ds(rbase + j * RC, RC)], prevv[b], semg[b])

        def wait_in(b):
            pltpu.make_async_copy(
                x_hbm.at[pl.ds(0, EC)], gat[b], semg[b]).wait()
            if has_prev:
                pltpu.make_async_copy(
                    prev_hbm.at[pl.ds(0, RC)], prevv[b], semg[b]).wait()

        def wait_out(b):
            pltpu.make_async_copy(
                outv[b], out_hbm.at[pl.ds(0, RC)], semo[b]).wait()

        def compute(j, b):
            nacc = 2 if (G <= 2 and deg >= 4) else 1

            def row(r, c2):
                ev = j * EC + r * deg
                eg = r * deg
                acc = [[None] * nacc for _ in range(G)]
                for d in range(deg):
                    wd = plsc.load_gather(
                        val_v, [lax.broadcast_in_dim(ev + d, (16,), ())])
                    a = d % nacc
                    for g in range(G):
                        term = wd * gat[b][eg + d, pl.ds(g * 16, 16)]
                        acc[g][a] = (term if acc[g][a] is None
                                     else acc[g][a] + term)
                for g in range(G):
                    res = acc[g][0]
                    for a in range(1, nacc):
                        res = res + acc[g][a]
                    if has_prev:
                        res = res + res - prevv[b][r, pl.ds(g * 16, 16)]
                    outv[b][r, pl.ds(g * 16, 16)] = res
                return c2

            lax.fori_loop(0, RC, row, 0)
            pltpu.async_copy(
                outv[b], out_hbm.at[pl.ds(rbase + j * RC, RC)], semo[b])

        if nch == 1:
            fire(0, 0)
            wait_in(0)
            compute(0, 0)
            wait_out(0)
        else:
            for b in range(NB - 1):
                fire(b, b)
            ngrp = nch // NB

            def grp(gi, c):
                j0 = gi * NB
                for b in range(NB):
                    j = j0 + b
                    nj = j + NB - 1

                    @pl.when(nj < nch)
                    def _(nj=nj, b=b):
                        fire(nj, (NB - 1 + b) % NB)
                    wait_in(b)

                    @pl.when(gi > 0)
                    def _(b=b):
                        wait_out(b)
                    compute(j, b)
                return c

            lax.fori_loop(0, ngrp, grp, 0)
            for b in range(NB):
                wait_out(b)

    return pl.kernel(
        body,
        out_type=jax.ShapeDtypeStruct((nrows, C), jnp.float32),
        mesh=mesh,
        scratch_types=scratch,
        compiler_params=pltpu.CompilerParams(needs_layout_passes=False,
                                             use_tc_tiling_on_sc=False),
    )


def _spmm(x, col, val, nrows, deg, prev=None):
    ncols, C = x.shape
    f = _make_spmm(ncols, nrows, C, deg, prev is not None)
    if prev is None:
        return f(x, col, val)
    return f(x, col, val, prev)


def _mm(ts, wbd, bias, add_in, act):
    n, ci = ts[0].shape
    kk, _, co = wbd.shape
    bn = min(n, 2048)
    grid = n // bn
    in_specs = [pl.BlockSpec((bn, ci), lambda i: (i, 0)) for _ in ts]
    in_specs.append(pl.BlockSpec((kk, ci, co), lambda i: (0, 0, 0)))
    in_specs.append(pl.BlockSpec((1, co), lambda i: (0, 0)))
    args = list(ts) + [wbd, bias]
    if add_in is not None:
        in_specs.append(pl.BlockSpec((bn, co), lambda i: (i, 0)))
        args.append(add_in)

    def body(*refs):
        if add_in is not None:
            *t_refs, w_ref, b_ref, a_ref, o_ref = refs
        else:
            *t_refs, w_ref, b_ref, o_ref = refs
            a_ref = None
        acc = jnp.dot(t_refs[0][...], w_ref[0],
                      preferred_element_type=jnp.float32)
        for k in range(1, len(t_refs)):
            acc = acc + jnp.dot(t_refs[k][...], w_ref[k],
                                preferred_element_type=jnp.float32)
        acc = acc + b_ref[...]
        if a_ref is not None:
            acc = acc + a_ref[...]
        if act == "relu":
            acc = jnp.maximum(acc, 0.0)
        elif act == "tanh":
            acc = jnp.tanh(acc)
        o_ref[...] = acc

    return pl.pallas_call(
        body,
        grid=(grid,),
        in_specs=in_specs,
        out_specs=pl.BlockSpec((bn, co), lambda i: (i, 0)),
        out_shape=jax.ShapeDtypeStruct((n, co), jnp.float32),
    )(*args)


def _fc_tc(emb, w_t, b):
    dout = w_t.shape[1]

    def body(e_ref, w_ref, b_ref, o_ref):
        acc = jnp.dot(e_ref[...], w_ref[...],
                      preferred_element_type=jnp.float32)
        o_ref[...] = jnp.maximum(acc + b_ref[...], 0.0)

    return pl.pallas_call(
        body,
        out_shape=jax.ShapeDtypeStruct((8, dout), jnp.float32),
    )(emb, w_t, b)


def _prep_w(p, fin, cout):
    fp, cp = _fp(fin), _fp(cout)
    w = p["W"]
    core = w.reshape(cout, _K, fin).transpose(1, 2, 0)
    if fin == 6:
        h1 = jnp.pad(core[:, :3, :], ((0, 0), (0, 1), (0, cp - cout)))
        h2 = jnp.pad(core[:, 3:, :], ((0, 0), (0, 1), (0, cp - cout)))
        wp = jnp.concatenate([h1, h2], axis=1)
    else:
        wp = jnp.pad(core, ((0, 0), (0, fp - fin), (0, cp - cout)))
    eye = jnp.eye(_B, dtype=jnp.float32)
    wbd = jnp.einsum("ij,kfc->kifjc", eye, wp).reshape(_K, _B * fp, _B * cp)
    bias = jnp.pad(p["b"], (0, cp - cout))
    bias = jnp.tile(bias, _B).reshape(1, _B * cp)
    return wbd, bias


def _cheb_ts(x, col, val, n):
    ts = [x]
    t1 = _spmm(x, col, val, n, 8)
    ts.append(t1)
    t0, t1_ = x, t1
    for _ in range(2, _K):
        t2 = _spmm(t1_, col, val, n, 8, prev=t0)
        ts.append(t2)
        t0, t1_ = t1_, t2
    return ts


def _cheb(x, wb, col, val, n, act=None, add_in=None):
    ts = _cheb_ts(x, col, val, n)
    return _mm(ts, wb[0], wb[1], add_in, act)


def _res(x, p, col, val, n, cin, cout):
    ts = _cheb_ts(x, col, val, n)
    if "shortcut" in p:
        wbd, bias = _prep_w(p["shortcut"], cin, cout)
        sc = _mm(ts, wbd, bias, None, None)
    else:
        sc = x
    w1, b1 = _prep_w(p["cheby1"], cin, cout)
    h = _mm(ts, w1, b1, None, "relu")
    ts2 = _cheb_ts(h, col, val, n)
    w2, b2 = _prep_w(p["cheby2"], cout, cout)
    return _mm(ts2, w2, b2, sc, "relu")


def kernel(image_emb, tex, params, lap_idx, lap_val, up_idx, up_val,
           down_idx, down_val):
    ns = [32768, 8192, 2048, 512, 128]

    emb_p = jnp.pad(image_emb, ((0, 8 - _B), (0, 0)))
    w_t = params["fc"]["W"].T
    b = params["fc"]["b"].reshape(1, -1)
    h = _fc_tc(emb_p, w_t, b)[:_B]
    x = h.reshape(_B, ns[4], 32).transpose(1, 0, 2).reshape(ns[4], _B * 32)

    x = _spmm(x, up_idx[3][1], up_val[3], ns[3], 3)
    x = _res(x, params["dec0"], lap_idx[3][1], lap_val[3], ns[3], 32, 16)
    x = _spmm(x, up_idx[2][1], up_val[2], ns[2], 3)
    x = _res(x, params["dec1"], lap_idx[2][1], lap_val[2], ns[2], 16, 16)
    x = _spmm(x, up_idx[1][1], up_val[1], ns[1], 3)
    x = _res(x, params["dec2"], lap_idx[1][1], lap_val[1], ns[1], 16, 16)
    x = _spmm(x, up_idx[0][1], up_val[0], ns[0], 3)
    dec = _res(x, params["dec3"], lap_idx[0][1], lap_val[0], ns[0], 16, 3)

    t = tex.transpose(1, 0, 2)
    t = jnp.pad(t, ((0, 0), (0, 0), (0, 1))).reshape(ns[0], _B * 4)
    r = _res(t, params["ref0"], lap_idx[0][1], lap_val[0], ns[0], 3, 16)
    r = _spmm(r, down_idx[1], down_val, ns[1], 4)
    r = _res(r, params["ref1"], lap_idx[1][1], lap_val[1], ns[1], 16, 32)
    r = _res(r, params["ref2"], lap_idx[1][1], lap_val[1], ns[1], 32, 32)
    r = _spmm(r, up_idx[0][1], up_val[0], ns[0], 3)
    r = _res(r, params["ref3"], lap_idx[0][1], lap_val[0], ns[0], 32, 16)
    ref = _res(r, params["ref4"], lap_idx[0][1], lap_val[0], ns[0], 16, 3)

    cat = jnp.concatenate(
        [dec.reshape(ns[0], _B, 4), ref.reshape(ns[0], _B, 4)],
        axis=2).reshape(ns[0], _B * 8)
    out = _cheb(cat, _prep_w(params["comb"], 6, 3),
                lap_idx[0][1], lap_val[0], ns[0], act="tanh")
    out = out.reshape(ns[0], _B, 4).transpose(1, 0, 2)[:, :, :3]
    return out


# speedup vs baseline: 23.2122x; 1.0086x over previous
import functools

import jax
import jax.numpy as jnp
from jax import lax
from jax.experimental import pallas as pl
from jax.experimental.pallas import tpu as pltpu
from jax.experimental.pallas import tpu_sc as plsc

_K = 6
_B = 4
_NW = 32
_LANES = 16


def _fp(f):
    return {3: 4, 6: 8}.get(f, f)


@functools.cache
def _make_spmm(ncols, nrows, C, deg, has_prev):
    epw = nrows * deg // _NW
    EC = min(192 if deg == 3 else 256, epw)
    assert EC % deg == 0 and epw % EC == 0 and EC % 8 == 0
    RC = EC // deg
    nch = epw // EC
    rows_pw = nrows // _NW
    G = C // _LANES
    assert C % _LANES == 0
    NB = min(2 if C > 64 else 4, nch)
    assert nch % NB == 0

    mesh = plsc.VectorSubcoreMesh(core_axis_name="c", subcore_axis_name="s",
                                  num_cores=2, num_subcores=16)
    scratch = [
        pltpu.VMEM((epw,), jnp.int32),
        pltpu.VMEM((epw,), jnp.float32),
    ]
    scratch += [pltpu.VMEM((EC, C), jnp.float32) for _ in range(NB)]
    scratch += [pltpu.VMEM((RC, C), jnp.float32) for _ in range(NB)]
    if has_prev:
        scratch += [pltpu.VMEM((RC, C), jnp.float32) for _ in range(NB)]
    scratch += [pltpu.SemaphoreType.DMA for _ in range(2 * NB)]

    def body(*refs):
        if has_prev:
            x_hbm, col_hbm, val_hbm, prev_hbm, out_hbm = refs[:5]
            s = refs[5:]
        else:
            x_hbm, col_hbm, val_hbm, out_hbm = refs[:4]
            s = refs[4:]
        col_v, val_v = s[0], s[1]
        p = 2
        gat = s[p:p + NB]; p += NB
        outv = s[p:p + NB]; p += NB
        if has_prev:
            prevv = s[p:p + NB]; p += NB
        semg = s[p:p + NB]
        semo = s[p + NB:p + 2 * NB]

        wid = lax.axis_index("s") * 2 + lax.axis_index("c")
        ebase = wid * epw
        rbase = wid * rows_pw
        pltpu.sync_copy(col_hbm.at[pl.ds(ebase, epw)], col_v)
        pltpu.sync_copy(val_hbm.at[pl.ds(ebase, epw)], val_v)

        def fire(j, b):
            pltpu.async_copy(
                x_hbm.at[col_v.at[pl.ds(j * EC, EC)]], gat[b], semg[b])
            if has_prev:
                pltpu.async_copy(
                    prev_hbm.at[pl.ds(rbase + j * RC, RC)], prevv[b], semg[b])

        def wait_in(b):
            pltpu.make_async_copy(
                x_hbm.at[pl.ds(0, EC)], gat[b], semg[b]).wait()
            if has_prev:
                pltpu.make_async_copy(
                    prev_hbm.at[pl.ds(0, RC)], prevv[b], semg[b]).wait()

        def wait_out(b):
            pltpu.make_async_copy(
                outv[b], out_hbm.at[pl.ds(0, RC)], semo[b]).wait()

        def compute(j, b):
            nacc = 2 if (G <= 2 and deg >= 4) else 1

            def row(r, c2):
                ev = j * EC + r * deg
                eg = r * deg
                acc = [[None] * nacc for _ in range(G)]
                for d in range(deg):
                    wd = plsc.load_gather(
                        val_v, [lax.broadcast_in_dim(ev + d, (16,), ())])
                    a = d % nacc
                    for g in range(G):
                        term = wd * gat[b][eg + d, pl.ds(g * 16, 16)]
                        acc[g][a] = (term if acc[g][a] is None
                                     else acc[g][a] + term)
                for g in range(G):
                    res = acc[g][0]
                    for a in range(1, nacc):
                        res = res + acc[g][a]
                    if has_prev:
                        res = res + res - prevv[b][r, pl.ds(g * 16, 16)]
                    outv[b][r, pl.ds(g * 16, 16)] = res
                return c2

            lax.fori_loop(0, RC, row, 0)
            pltpu.async_copy(
                outv[b], out_hbm.at[pl.ds(rbase + j * RC, RC)], semo[b])

        if nch == 1:
            fire(0, 0)
            wait_in(0)
            compute(0, 0)
            wait_out(0)
        else:
            for b in range(NB - 1):
                fire(b, b)
            ngrp = nch // NB

            def grp(gi, c):
                j0 = gi * NB
                for b in range(NB):
                    j = j0 + b
                    nj = j + NB - 1

                    @pl.when(nj < nch)
                    def _(nj=nj, b=b):
                        fire(nj, (NB - 1 + b) % NB)
                    wait_in(b)

                    @pl.when(gi > 0)
                    def _(b=b):
                        wait_out(b)
                    compute(j, b)
                return c

            lax.fori_loop(0, ngrp, grp, 0)
            for b in range(NB):
                wait_out(b)

    return pl.kernel(
        body,
        out_type=jax.ShapeDtypeStruct((nrows, C), jnp.float32),
        mesh=mesh,
        scratch_types=scratch,
        compiler_params=pltpu.CompilerParams(needs_layout_passes=False,
                                             use_tc_tiling_on_sc=False),
    )


def _spmm(x, col, val, nrows, deg, prev=None):
    ncols, C = x.shape
    f = _make_spmm(ncols, nrows, C, deg, prev is not None)
    if prev is None:
        return f(x, col, val)
    return f(x, col, val, prev)


def _mm(ts, wbd, bias, add_in, act):
    n, ci = ts[0].shape
    kk, _, co = wbd.shape
    bn = min(n, 2048)
    grid = n // bn
    in_specs = [pl.BlockSpec((bn, ci), lambda i: (i, 0)) for _ in ts]
    in_specs.append(pl.BlockSpec((kk, ci, co), lambda i: (0, 0, 0)))
    in_specs.append(pl.BlockSpec((1, co), lambda i: (0, 0)))
    args = list(ts) + [wbd, bias]
    if add_in is not None:
        in_specs.append(pl.BlockSpec((bn, co), lambda i: (i, 0)))
        args.append(add_in)

    def body(*refs):
        if add_in is not None:
            *t_refs, w_ref, b_ref, a_ref, o_ref = refs
        else:
            *t_refs, w_ref, b_ref, o_ref = refs
            a_ref = None
        acc = jnp.dot(t_refs[0][...], w_ref[0],
                      preferred_element_type=jnp.float32)
        for k in range(1, len(t_refs)):
            acc = acc + jnp.dot(t_refs[k][...], w_ref[k],
                                preferred_element_type=jnp.float32)
        acc = acc + b_ref[...]
        if a_ref is not None:
            acc = acc + a_ref[...]
        if act == "relu":
            acc = jnp.maximum(acc, 0.0)
        elif act == "tanh":
            acc = jnp.tanh(acc)
        o_ref[...] = acc

    return pl.pallas_call(
        body,
        grid=(grid,),
        in_specs=in_specs,
        out_specs=pl.BlockSpec((bn, co), lambda i: (i, 0)),
        out_shape=jax.ShapeDtypeStruct((n, co), jnp.float32),
    )(*args)


def _fc_tc(emb, w_t, b):
    dout = w_t.shape[1]

    def body(e_ref, w_ref, b_ref, o_ref):
        acc = jnp.dot(e_ref[...], w_ref[...],
                      preferred_element_type=jnp.float32)
        o_ref[...] = jnp.maximum(acc + b_ref[...], 0.0)

    return pl.pallas_call(
        body,
        out_shape=jax.ShapeDtypeStruct((8, dout), jnp.float32),
    )(emb, w_t, b)


def _prep_w(p, fin, cout):
    fp, cp = _fp(fin), _fp(cout)
    w = p["W"]
    core = w.reshape(cout, _K, fin).transpose(1, 2, 0)
    if fin == 6:
        h1 = jnp.pad(core[:, :3, :], ((0, 0), (0, 1), (0, cp - cout)))
        h2 = jnp.pad(core[:, 3:, :], ((0, 0), (0, 1), (0, cp - cout)))
        wp = jnp.concatenate([h1, h2], axis=1)
    else:
        wp = jnp.pad(core, ((0, 0), (0, fp - fin), (0, cp - cout)))
    eye = jnp.eye(_B, dtype=jnp.float32)
    wbd = jnp.einsum("ij,kfc->kifjc", eye, wp).reshape(_K, _B * fp, _B * cp)
    bias = jnp.pad(p["b"], (0, cp - cout))
    bias = jnp.tile(bias, _B).reshape(1, _B * cp)
    return wbd, bias


def _cheb_ts(x, col, val, n):
    ts = [x]
    t1 = _spmm(x, col, val, n, 8)
    ts.append(t1)
    t0, t1_ = x, t1
    for _ in range(2, _K):
        t2 = _spmm(t1_, col, val, n, 8, prev=t0)
        ts.append(t2)
        t0, t1_ = t1_, t2
    return ts


def _cheb(x, wb, col, val, n, act=None, add_in=None):
    ts = _cheb_ts(x, col, val, n)
    return _mm(ts, wb[0], wb[1], add_in, act)


def _res(x, p, col, val, n, cin, cout):
    ts = _cheb_ts(x, col, val, n)
    if "shortcut" in p:
        wbd, bias = _prep_w(p["shortcut"], cin, cout)
        sc = _mm(ts, wbd, bias, None, None)
    else:
        sc = x
    w1, b1 = _prep_w(p["cheby1"], cin, cout)
    h = _mm(ts, w1, b1, None, "relu")
    ts2 = _cheb_ts(h, col, val, n)
    w2, b2 = _prep_w(p["cheby2"], cout, cout)
    return _mm(ts2, w2, b2, sc, "relu")


def kernel(image_emb, tex, params, lap_idx, lap_val, up_idx, up_val,
           down_idx, down_val):
    ns = [32768, 8192, 2048, 512, 128]

    emb_p = jnp.pad(image_emb, ((0, 8 - _B), (0, 0)))
    w_t = params["fc"]["W"].T
    b = params["fc"]["b"].reshape(1, -1)
    h = _fc_tc(emb_p, w_t, b)[:_B]
    x = h.reshape(_B, ns[4], 32).transpose(1, 0, 2).reshape(ns[4], _B * 32)

    x = _spmm(x, up_idx[3][1], up_val[3], ns[3], 3)
    x = _res(x, params["dec0"], lap_idx[3][1], lap_val[3], ns[3], 32, 16)
    x = _spmm(x, up_idx[2][1], up_val[2], ns[2], 3)
    x = _res(x, params["dec1"], lap_idx[2][1], lap_val[2], ns[2], 16, 16)
    x = _spmm(x, up_idx[1][1], up_val[1], ns[1], 3)
    x = _res(x, params["dec2"], lap_idx[1][1], lap_val[1], ns[1], 16, 16)
    x = _spmm(x, up_idx[0][1], up_val[0], ns[0], 3)
    dec = _res(x, params["dec3"], lap_idx[0][1], lap_val[0], ns[0], 16, 3)

    t = tex.transpose(1, 0, 2)
    t = jnp.pad(t, ((0, 0), (0, 0), (0, 1))).reshape(ns[0], _B * 4)
    r = _res(t, params["ref0"], lap_idx[0][1], lap_val[0], ns[0], 3, 16)
    r = _spmm(r, down_idx[1], down_val, ns[1], 4)
    r = _res(r, params["ref1"], lap_idx[1][1], lap_val[1], ns[1], 16, 32)
    r = _res(r, params["ref2"], lap_idx[1][1], lap_val[1], ns[1], 32, 32)
    r = _spmm(r, up_idx[0][1], up_val[0], ns[0], 3)
    r = _res(r, params["ref3"], lap_idx[0][1], lap_val[0], ns[0], 32, 16)
    ref = _res(r, params["ref4"], lap_idx[0][1], lap_val[0], ns[0], 16, 3)

    cat = jnp.concatenate(
        [dec.reshape(ns[0], _B, 4), ref.reshape(ns[0], _B, 4)],
        axis=2).reshape(ns[0], _B * 8)
    out = _cheb(cat, _prep_w(params["comb"], 6, 3),
                lap_idx[0][1], lap_val[0], ns[0], act="tanh")
    out = out.reshape(ns[0], _B, 4).transpose(1, 0, 2)[:, :, :3]
    return out


# speedup vs baseline: 24.2398x; 1.0443x over previous
import functools

import jax
import jax.numpy as jnp
from jax import lax
from jax.experimental import pallas as pl
from jax.experimental.pallas import tpu as pltpu
from jax.experimental.pallas import tpu_sc as plsc

_K = 6
_B = 4
_NW = 32
_LANES = 16


def _fp(f):
    return {3: 4, 6: 8}.get(f, f)


@functools.cache
def _make_spmm(ncols, nrows, C, deg, has_prev):
    epw = nrows * deg // _NW
    EC = min(192 if deg == 3 else 256, epw)
    assert EC % deg == 0 and epw % EC == 0 and EC % 8 == 0
    RC = EC // deg
    nch = epw // EC
    rows_pw = nrows // _NW
    G = C // _LANES
    assert C % _LANES == 0
    NB = min(2 if C > 64 else 4, nch)
    assert nch % NB == 0

    mesh = plsc.VectorSubcoreMesh(core_axis_name="c", subcore_axis_name="s",
                                  num_cores=2, num_subcores=16)
    scratch = [
        pltpu.VMEM((epw,), jnp.int32),
        pltpu.VMEM((epw,), jnp.float32),
    ]
    scratch += [pltpu.VMEM((EC, C), jnp.float32) for _ in range(NB)]
    scratch += [pltpu.VMEM((RC, C), jnp.float32) for _ in range(NB)]
    if has_prev:
        scratch += [pltpu.VMEM((RC, C), jnp.float32) for _ in range(NB)]
    scratch += [pltpu.SemaphoreType.DMA for _ in range(2 * NB)]

    def body(*refs):
        if has_prev:
            x_hbm, col_hbm, val_hbm, prev_hbm, out_hbm = refs[:5]
            s = refs[5:]
        else:
            x_hbm, col_hbm, val_hbm, out_hbm = refs[:4]
            s = refs[4:]
        col_v, val_v = s[0], s[1]
        p = 2
        gat = s[p:p + NB]; p += NB
        outv = s[p:p + NB]; p += NB
        if has_prev:
            prevv = s[p:p + NB]; p += NB
        semg = s[p:p + NB]
        semo = s[p + NB:p + 2 * NB]

        wid = lax.axis_index("s") * 2 + lax.axis_index("c")
        ebase = wid * epw
        rbase = wid * rows_pw
        pltpu.sync_copy(col_hbm.at[pl.ds(ebase, epw)], col_v)
        pltpu.sync_copy(val_hbm.at[pl.ds(ebase, epw)], val_v)

        def fire(j, b):
            pltpu.async_copy(
                x_hbm.at[col_v.at[pl.ds(j * EC, EC)]], gat[b], semg[b])
            if has_prev:
                pltpu.async_copy(
                    prev_hbm.at[pl.ds(rbase + j * RC, RC)], prevv[b], semg[b])

        def wait_in(b):
            pltpu.make_async_copy(
                x_hbm.at[pl.ds(0, EC)], gat[b], semg[b]).wait()
            if has_prev:
                pltpu.make_async_copy(
                    prev_hbm.at[pl.ds(0, RC)], prevv[b], semg[b]).wait()

        def wait_out(b):
            pltpu.make_async_copy(
                outv[b], out_hbm.at[pl.ds(0, RC)], semo[b]).wait()

        def compute(j, b):
            nacc = 2 if (G <= 2 and deg >= 4) else 1

            def row(r, c2):
                ev = j * EC + r * deg
                eg = r * deg
                acc = [[None] * nacc for _ in range(G)]
                for d in range(deg):
                    wd = plsc.load_gather(
                        val_v, [lax.broadcast_in_dim(ev + d, (16,), ())])
                    a = d % nacc
                    for g in range(G):
                        term = wd * gat[b][eg + d, pl.ds(g * 16, 16)]
                        acc[g][a] = (term if acc[g][a] is None
                                     else acc[g][a] + term)
                for g in range(G):
                    res = acc[g][0]
                    for a in range(1, nacc):
                        res = res + acc[g][a]
                    if has_prev:
                        res = res + res - prevv[b][r, pl.ds(g * 16, 16)]
                    outv[b][r, pl.ds(g * 16, 16)] = res
                return c2

            lax.fori_loop(0, RC, row, 0)
            pltpu.async_copy(
                outv[b], out_hbm.at[pl.ds(rbase + j * RC, RC)], semo[b])

        if nch == 1:
            fire(0, 0)
            wait_in(0)
            compute(0, 0)
            wait_out(0)
        else:
            for b in range(NB - 1):
                fire(b, b)
            ngrp = nch // NB

            def grp(gi, c):
                j0 = gi * NB
                for b in range(NB):
                    j = j0 + b
                    nj = j + NB - 1

                    @pl.when(nj < nch)
                    def _(nj=nj, b=b):
                        fire(nj, (NB - 1 + b) % NB)
                    wait_in(b)

                    @pl.when(gi > 0)
                    def _(b=b):
                        wait_out(b)
                    compute(j, b)
                return c

            lax.fori_loop(0, ngrp, grp, 0)
            for b in range(NB):
                wait_out(b)

    return pl.kernel(
        body,
        out_type=jax.ShapeDtypeStruct((nrows, C), jnp.float32),
        mesh=mesh,
        scratch_types=scratch,
        compiler_params=pltpu.CompilerParams(needs_layout_passes=False,
                                             use_tc_tiling_on_sc=False),
    )


def _spmm(x, col, val, nrows, deg, prev=None):
    ncols, C = x.shape
    f = _make_spmm(ncols, nrows, C, deg, prev is not None)
    if prev is None:
        return f(x, col, val)
    return f(x, col, val, prev)


def _mm_multi(ts, heads):
    n, ci = ts[0].shape
    kk = heads[0][0].shape[0]
    bn = min(n, 2048)
    grid = n // bn
    nt = len(ts)
    in_specs = [pl.BlockSpec((bn, ci), lambda i: (i, 0)) for _ in ts]
    args = list(ts)
    layout = []
    for wbd, bias, add_in, act in heads:
        co = wbd.shape[2]
        in_specs.append(pl.BlockSpec((kk, ci, co), lambda i: (0, 0, 0)))
        in_specs.append(pl.BlockSpec((1, co), lambda i: (0, 0)))
        args += [wbd, bias]
        if add_in is not None:
            in_specs.append(pl.BlockSpec((bn, co), lambda i: (i, 0)))
            args.append(add_in)
        layout.append((co, add_in is not None, act))

    def body(*refs):
        t_refs = refs[:nt]
        pos = nt
        o_refs = refs[len(refs) - len(heads):]
        for h, (co, has_add, act) in enumerate(layout):
            w_ref = refs[pos]; b_ref = refs[pos + 1]
            pos_h = pos + 2
            a_ref = refs[pos_h] if has_add else None
            pos = pos_h + (1 if has_add else 0)
            acc = jnp.dot(t_refs[0][...], w_ref[0],
                          preferred_element_type=jnp.float32)
            for k in range(1, nt):
                acc = acc + jnp.dot(t_refs[k][...], w_ref[k],
                                    preferred_element_type=jnp.float32)
            acc = acc + b_ref[...]
            if a_ref is not None:
                acc = acc + a_ref[...]
            if act == "relu":
                acc = jnp.maximum(acc, 0.0)
            elif act == "tanh":
                acc = jnp.tanh(acc)
            o_refs[h][...] = acc

    outs = pl.pallas_call(
        body,
        grid=(grid,),
        in_specs=in_specs,
        out_specs=[pl.BlockSpec((bn, co), lambda i: (i, 0))
                   for co, _, _ in layout],
        out_shape=[jax.ShapeDtypeStruct((n, co), jnp.float32)
                   for co, _, _ in layout],
    )(*args)
    return outs


def _mm(ts, wbd, bias, add_in, act):
    return _mm_multi(ts, [(wbd, bias, add_in, act)])[0]


def _fc_tc(emb, w_t, b):
    dout = w_t.shape[1]

    def body(e_ref, w_ref, b_ref, o_ref):
        acc = jnp.dot(e_ref[...], w_ref[...],
                      preferred_element_type=jnp.float32)
        o_ref[...] = jnp.maximum(acc + b_ref[...], 0.0)

    return pl.pallas_call(
        body,
        out_shape=jax.ShapeDtypeStruct((8, dout), jnp.float32),
    )(emb, w_t, b)


def _prep_w(p, fin, cout):
    fp, cp = _fp(fin), _fp(cout)
    w = p["W"]
    core = w.reshape(cout, _K, fin).transpose(1, 2, 0)
    if fin == 6:
        h1 = jnp.pad(core[:, :3, :], ((0, 0), (0, 1), (0, cp - cout)))
        h2 = jnp.pad(core[:, 3:, :], ((0, 0), (0, 1), (0, cp - cout)))
        wp = jnp.concatenate([h1, h2], axis=1)
    else:
        wp = jnp.pad(core, ((0, 0), (0, fp - fin), (0, cp - cout)))
    eye = jnp.eye(_B, dtype=jnp.float32)
    wbd = jnp.einsum("ij,kfc->kifjc", eye, wp).reshape(_K, _B * fp, _B * cp)
    bias = jnp.pad(p["b"], (0, cp - cout))
    bias = jnp.tile(bias, _B).reshape(1, _B * cp)
    return wbd, bias


def _cheb_ts(x, col, val, n):
    ts = [x]
    t1 = _spmm(x, col, val, n, 8)
    ts.append(t1)
    t0, t1_ = x, t1
    for _ in range(2, _K):
        t2 = _spmm(t1_, col, val, n, 8, prev=t0)
        ts.append(t2)
        t0, t1_ = t1_, t2
    return ts


def _cheb(x, wb, col, val, n, act=None, add_in=None):
    ts = _cheb_ts(x, col, val, n)
    return _mm(ts, wb[0], wb[1], add_in, act)


def _res(x, p, col, val, n, cin, cout):
    ts = _cheb_ts(x, col, val, n)
    w1, b1 = _prep_w(p["cheby1"], cin, cout)
    if "shortcut" in p:
        wbd, bias = _prep_w(p["shortcut"], cin, cout)
        sc, h = _mm_multi(ts, [(wbd, bias, None, None),
                               (w1, b1, None, "relu")])
    else:
        sc = x
        h = _mm(ts, w1, b1, None, "relu")
    ts2 = _cheb_ts(h, col, val, n)
    w2, b2 = _prep_w(p["cheby2"], cout, cout)
    return _mm(ts2, w2, b2, sc, "relu")


def kernel(image_emb, tex, params, lap_idx, lap_val, up_idx, up_val,
           down_idx, down_val):
    ns = [32768, 8192, 2048, 512, 128]

    emb_p = jnp.pad(image_emb, ((0, 8 - _B), (0, 0)))
    w_t = params["fc"]["W"].T
    b = params["fc"]["b"].reshape(1, -1)
    h = _fc_tc(emb_p, w_t, b)[:_B]
    x = h.reshape(_B, ns[4], 32).transpose(1, 0, 2).reshape(ns[4], _B * 32)

    x = _spmm(x, up_idx[3][1], up_val[3], ns[3], 3)
    x = _res(x, params["dec0"], lap_idx[3][1], lap_val[3], ns[3], 32, 16)
    x = _spmm(x, up_idx[2][1], up_val[2], ns[2], 3)
    x = _res(x, params["dec1"], lap_idx[2][1], lap_val[2], ns[2], 16, 16)
    x = _spmm(x, up_idx[1][1], up_val[1], ns[1], 3)
    x = _res(x, params["dec2"], lap_idx[1][1], lap_val[1], ns[1], 16, 16)
    x = _spmm(x, up_idx[0][1], up_val[0], ns[0], 3)
    dec = _res(x, params["dec3"], lap_idx[0][1], lap_val[0], ns[0], 16, 3)

    t = tex.transpose(1, 0, 2)
    t = jnp.pad(t, ((0, 0), (0, 0), (0, 1))).reshape(ns[0], _B * 4)
    r = _res(t, params["ref0"], lap_idx[0][1], lap_val[0], ns[0], 3, 16)
    r = _spmm(r, down_idx[1], down_val, ns[1], 4)
    r = _res(r, params["ref1"], lap_idx[1][1], lap_val[1], ns[1], 16, 32)
    r = _res(r, params["ref2"], lap_idx[1][1], lap_val[1], ns[1], 32, 32)
    r = _spmm(r, up_idx[0][1], up_val[0], ns[0], 3)
    r = _res(r, params["ref3"], lap_idx[0][1], lap_val[0], ns[0], 32, 16)
    ref = _res(r, params["ref4"], lap_idx[0][1], lap_val[0], ns[0], 16, 3)

    cat = jnp.concatenate(
        [dec.reshape(ns[0], _B, 4), ref.reshape(ns[0], _B, 4)],
        axis=2).reshape(ns[0], _B * 8)
    out = _cheb(cat, _prep_w(params["comb"], 6, 3),
                lap_idx[0][1], lap_val[0], ns[0], act="tanh")
    out = out.reshape(ns[0], _B, 4).transpose(1, 0, 2)[:, :, :3]
    return out
